# baseline (speedup 1.0000x reference)
import jax
import jax.numpy as jnp
from jax.experimental import pallas as pl


def _relu_pallas(x):
    def body(x_ref, o_ref):
        o_ref[...] = jnp.maximum(x_ref[...], 0.0)
    return pl.pallas_call(
        body,
        out_shape=jax.ShapeDtypeStruct(x.shape, x.dtype),
    )(x)


def kernel(inputs, edge_index, batch, edge_weight, W0, W1, W2, W3, conv_bias,
           gn_weight, gn_bias, gn_mean_scale):
    Ws = [W0, W1, W2, W3]
    n = inputs.shape[0]
    G = 64
    H = W0.shape[1]
    K = 3
    loop = jnp.arange(n, dtype=edge_index.dtype)
    row = jnp.concatenate([edge_index[0], loop])
    col = jnp.concatenate([edge_index[1], loop])
    ew = jnp.concatenate([edge_weight, jnp.ones((n,), dtype=edge_weight.dtype)])
    deg = jnp.zeros((n,), inputs.dtype).at[col].add(ew)
    dinv = jnp.where(deg > 0, jax.lax.rsqrt(jnp.maximum(deg, 1e-12)), 0.0)
    norm = dinv[row] * ew * dinv[col]
    out = inputs @ Ws[0]
    x = inputs
    for k in range(1, K + 1):
        msg = norm[:, None] * x[row]
        x = jnp.zeros((n, x.shape[1]), x.dtype).at[col].add(msg)
        out = out + x @ Ws[k]
    out = out + conv_bias
    cnt = jnp.zeros((G,), out.dtype).at[batch].add(1.0)
    cnt = jnp.maximum(cnt, 1.0)
    mean = jnp.zeros((G, H), out.dtype).at[batch].add(out) / cnt[:, None]
    centered = out - mean[batch] * gn_mean_scale
    var = jnp.zeros((G, H), out.dtype).at[batch].add(centered * centered) / cnt[:, None]
    std = jnp.sqrt(var[batch] + 1e-5)
    h = gn_weight * centered / std + gn_bias
    h = _relu_pallas(h)
    gmean = jnp.zeros((G, H), h.dtype).at[batch].add(h) / cnt[:, None]
    gmax = jnp.full((G, H), -jnp.inf, h.dtype).at[batch].max(h)
    gmax = jnp.where(jnp.isfinite(gmax), gmax, 0.0)
    flat = jnp.concatenate([gmean, gmax], axis=-1)
    return (h, flat, edge_index, edge_weight, batch)


# speedup vs baseline: 14.2601x; 14.2601x over previous
import jax
import jax.numpy as jnp
from jax import lax
from jax.experimental import pallas as pl
from jax.experimental.pallas import tpu as pltpu
from jax.experimental.pallas import tpu_sc as plsc

N = 10000
NP = 10240
E = 320000
D = 128
H = 128
G = 64
GP = 72
NC = 2
NS = 16
NW = NC * NS
EC = 80
EPW = E // NW
ECH = EPW // EC
RPW = NP // NW
RCH = RPW // EC
RPT = NP // NS
APT = N // NS

_mesh = plsc.VectorSubcoreMesh(core_axis_name="c", subcore_axis_name="s")
_f32 = jnp.float32
_sc_params = pltpu.CompilerParams(needs_layout_passes=False)


def _wid():
    return lax.axis_index("c") * NS + lax.axis_index("s")


def _zero_vec():
    return jnp.zeros((16,), _f32)


def _deg_body(coli3, ew2, batch2, degp, cntp, dacc, cacc, colbuf2, ewbuf,
              onesbuf, bibuf2, zbuf):
    cid = lax.axis_index("c")
    sid = lax.axis_index("s")
    wid = _wid()

    def _z(i, _):
        zbuf[pl.ds(i * 16, 16)] = _zero_vec()
        return 0
    lax.fori_loop(0, RPT // 16, _z, 0)

    def _o(i, _):
        onesbuf[pl.ds(i * 16, 16)] = jnp.ones((16,), _f32)
        return 0
    lax.fori_loop(0, EC // 16, _o, 0)

    pltpu.sync_copy(zbuf, dacc.at[pl.ds(sid * RPT, RPT)])

    @pl.when(sid == 0)
    def _():
        pltpu.sync_copy(zbuf.at[pl.ds(0, GP)], cacc)

    pltpu.sync_copy(coli3.at[wid], colbuf2)
    pltpu.sync_copy(ew2.at[wid], ewbuf)
    pltpu.sync_copy(batch2.at[wid], bibuf2)

    plsc.subcore_barrier()

    def _edges(c, _):
        pltpu.sync_copy(ewbuf.at[pl.ds(c * EC, EC)], dacc.at[colbuf2.at[c]],
                        add=True)
        return 0
    lax.fori_loop(0, ECH, _edges, 0)

    def _nodes(c, _):
        pltpu.sync_copy(onesbuf, cacc.at[bibuf2.at[c]], add=True)
        return 0
    lax.fori_loop(0, RCH, _nodes, 0)

    plsc.subcore_barrier()

    pltpu.sync_copy(dacc.at[pl.ds(sid * RPT, RPT)],
                    degp.at[cid, pl.ds(sid * RPT, RPT)])

    @pl.when(sid == 0)
    def _():
        pltpu.sync_copy(cacc, cntp.at[cid])


_deg_sc = pl.kernel(
    _deg_body,
    out_type=(jax.ShapeDtypeStruct((NC, NP), _f32),
              jax.ShapeDtypeStruct((NC, GP), _f32)),
    mesh=_mesh,
    compiler_params=_sc_params,
    scratch_types=[
        pltpu.VMEM_SHARED((NP,), _f32),
        pltpu.VMEM_SHARED((GP,), _f32),
        pltpu.VMEM((ECH, EC), jnp.int32),
        pltpu.VMEM((EPW,), _f32),
        pltpu.VMEM((EC,), _f32),
        pltpu.VMEM((RCH, EC), jnp.int32),
        pltpu.VMEM((RPT,), _f32),
    ],
)


def _norm_body(rowi2, coli2, ew2, dinv, normo, dvb, rib, cib, nbuf):
    wid = _wid()
    pltpu.sync_copy(dinv, dvb)
    pltpu.sync_copy(rowi2.at[wid], rib)
    pltpu.sync_copy(coli2.at[wid], cib)
    pltpu.sync_copy(ew2.at[wid], nbuf)

    def _chunk(t, _):
        sl = pl.ds(t * 16, 16)
        dr = plsc.load_gather(dvb, [rib[sl]])
        dc = plsc.load_gather(dvb, [cib[sl]])
        nbuf[sl] = dr * nbuf[sl] * dc
        return 0
    lax.fori_loop(0, EPW // 16, _chunk, 0)
    pltpu.sync_copy(nbuf, normo.at[wid])


_norm_sc = pl.kernel(
    _norm_body,
    out_type=jax.ShapeDtypeStruct((NW, EPW), _f32),
    mesh=_mesh,
    compiler_params=_sc_params,
    scratch_types=[
        pltpu.VMEM((NP,), _f32),
        pltpu.VMEM((EPW,), jnp.int32),
        pltpu.VMEM((EPW,), jnp.int32),
        pltpu.VMEM((EPW,), _f32),
    ],
)


HC = 40
HCH = EPW // HC


def _scale_chunk(rows_b, nbuf, c):
    base = c * HC
    nvA = nbuf[pl.ds(base, 16)]
    nvB = nbuf[pl.ds(base + 16, 16)]
    nvC = nbuf[pl.ds(base + 24, 16)]
    for e in range(HC):
        if e < 16:
            nv = nvA[e]
        elif e < 32:
            nv = nvB[e - 16]
        else:
            nv = nvC[e - 24]
        nvec = jnp.full((16,), nv, _f32)
        for j in range(8):
            sl = pl.ds(j * 16, 16)
            rows_b[e, sl] = rows_b[e, sl] * nvec


def _stage_cols(csm, cib, c):
    base = c * HC
    csm[pl.ds(0, 16)] = cib[pl.ds(base, 16)]
    csm[pl.ds(16, 16)] = cib[pl.ds(base + 16, 16)]
    csm[pl.ds(24, 16)] = cib[pl.ds(base + 24, 16)]


def _hop_body(s, rowi2, coli2, norm2, zrows, outp, acc, rib, cib, nbuf,
              csm0, csm1, rows0, rows1, gsem0, gsem1):
    cid = lax.axis_index("c")
    sid = lax.axis_index("s")
    wid = _wid()

    pltpu.sync_copy(zrows.at[pl.ds(0, 624)], acc.at[pl.ds(sid * 624, 624)])

    @pl.when(sid == NS - 1)
    def _():
        pltpu.sync_copy(zrows.at[pl.ds(624, 16)], acc.at[pl.ds(9984, 16)])

    pltpu.sync_copy(rowi2.at[wid], rib)
    pltpu.sync_copy(coli2.at[wid], cib)
    pltpu.sync_copy(norm2.at[wid], nbuf)

    plsc.subcore_barrier()

    pltpu.async_copy(s.at[rib.at[pl.ds(0, HC)]], rows0, gsem0)

    def _pair(cc, _):
        c0 = cc * 2
        c1 = c0 + 1
        pltpu.make_async_copy(s.at[rib.at[pl.ds(c0 * HC, HC)]], rows0,
                              gsem0).wait()
        d1 = pltpu.async_copy(s.at[rib.at[pl.ds(c1 * HC, HC)]], rows1,
                              gsem1)
        _scale_chunk(rows0, nbuf, c0)
        _stage_cols(csm0, cib, c0)
        pltpu.sync_copy(rows0, acc.at[csm0], add=True)
        d1.wait()

        @pl.when(cc < HCH // 2 - 1)
        def _():
            pltpu.async_copy(s.at[rib.at[pl.ds((c0 + 2) * HC, HC)]], rows0,
                             gsem0)

        _scale_chunk(rows1, nbuf, c1)
        _stage_cols(csm1, cib, c1)
        pltpu.sync_copy(rows1, acc.at[csm1], add=True)
        return 0
    lax.fori_loop(0, HCH // 2, _pair, 0)

    plsc.subcore_barrier()
    pltpu.sync_copy(acc.at[pl.ds(sid * 624, 624)],
                    outp.at[cid, pl.ds(sid * 624, 624)])

    @pl.when(sid == NS - 1)
    def _():
        pltpu.sync_copy(acc.at[pl.ds(9984, 16)],
                        outp.at[cid, pl.ds(9984, 16)])


_hop_sc = pl.kernel(
    _hop_body,
    out_type=jax.ShapeDtypeStruct((NC, N, D), _f32),
    mesh=_mesh,
    compiler_params=_sc_params,
    scratch_types=[
        pltpu.VMEM_SHARED((N, D), _f32),
        pltpu.VMEM((EPW,), jnp.int32),
        pltpu.VMEM((EPW,), jnp.int32),
        pltpu.VMEM((EPW,), _f32),
        pltpu.VMEM((HC,), jnp.int32),
        pltpu.VMEM((HC,), jnp.int32),
        pltpu.VMEM((HC, D), _f32),
        pltpu.VMEM((HC, D), _f32),
        pltpu.SemaphoreType.DMA,
        pltpu.SemaphoreType.DMA,
    ],
)


def _pool_body(outn, batch2, Atab, Btab, ho, hsp, gmp, hsacc, At, Bt, gmax_t,
               rbuf, hbuf, bibuf2):
    cid = lax.axis_index("c")
    sid = lax.axis_index("s")
    wid = _wid()

    pltpu.sync_copy(Atab, At)
    pltpu.sync_copy(Btab, Bt)
    pltpu.sync_copy(batch2.at[wid], bibuf2)

    def _im(i, _):
        for j in range(8):
            gmax_t[i, pl.ds(j * 16, 16)] = jnp.full((16,), -jnp.inf, _f32)
        return 0
    lax.fori_loop(0, GP, _im, 0)

    def _zh(i, _):
        for j in range(8):
            hbuf[i, pl.ds(j * 16, 16)] = _zero_vec()
        return 0
    lax.fori_loop(0, GP, _zh, 0)

    @pl.when(sid == 0)
    def _():
        pltpu.sync_copy(hbuf.at[pl.ds(0, GP)], hsacc)

    plsc.subcore_barrier()

    def _chunk(c, _):
        base = wid * RPW + c * EC
        pltpu.sync_copy(outn.at[pl.ds(base, EC)], rbuf)

        def _node(t, _):
            b16 = bibuf2[c, pl.ds(t * 16, 16)]
            for e in range(16):
                b = b16[e]
                i = t * 16 + e
                for j in range(8):
                    sl = pl.ds(j * 16, 16)
                    hv = jnp.maximum(rbuf[i, sl] * At[b, sl] + Bt[b, sl],
                                     0.0)
                    hbuf[i, sl] = hv
                    gmax_t[b, sl] = jnp.maximum(gmax_t[b, sl], hv)
            return 0
        lax.fori_loop(0, EC // 16, _node, 0)

        pltpu.sync_copy(hbuf, ho.at[pl.ds(base, EC)])
        pltpu.sync_copy(hbuf, hsacc.at[bibuf2.at[c]], add=True)
        return 0
    lax.fori_loop(0, RCH, _chunk, 0)

    plsc.subcore_barrier()

    @pl.when(sid == 0)
    def _():
        pltpu.sync_copy(hsacc, hsp.at[cid])
    pltpu.sync_copy(gmax_t, gmp.at[wid])


_pool_sc = pl.kernel(
    _pool_body,
    out_type=(jax.ShapeDtypeStruct((NP, H), _f32),
              jax.ShapeDtypeStruct((NC, GP, H), _f32),
              jax.ShapeDtypeStruct((NW, GP, H), _f32)),
    mesh=_mesh,
    compiler_params=_sc_params,
    scratch_types=[
        pltpu.VMEM_SHARED((GP, H), _f32),
        pltpu.VMEM((GP, H), _f32),
        pltpu.VMEM((GP, H), _f32),
        pltpu.VMEM((GP, H), _f32),
        pltpu.VMEM((EC, H), _f32),
        pltpu.VMEM((EC, H), _f32),
        pltpu.VMEM((RCH, EC), jnp.int32),
    ],
)


_BLK = 2000
_NB = N // _BLK


def _z_body(x_ref, w_ref, o_ref):
    o_ref[...] = jnp.dot(x_ref[...], w_ref[...],
                         preferred_element_type=_f32)


def _matmul_z(x, wcat):
    return pl.pallas_call(
        _z_body,
        grid=(_NB,),
        in_specs=[pl.BlockSpec((_BLK, D), lambda i: (i, 0)),
                  pl.BlockSpec((D, 4 * H), lambda i: (0, 0))],
        out_specs=pl.BlockSpec((_BLK, 4 * H), lambda i: (i, 0)),
        out_shape=jax.ShapeDtypeStruct((N, 4 * H), _f32),
    )(x, wcat)


def _comb_body(p0_ref, p1_ref, s_ref, z_ref, sw_ref, o_ref):
    o_ref[...] = (p0_ref[0] + p1_ref[0] + sw_ref[...] * s_ref[...]
                  + z_ref[...])


def _combine_tc(partial, s, Z, kcol, selfw2d):
    return pl.pallas_call(
        _comb_body,
        grid=(_NB,),
        in_specs=[pl.BlockSpec((1, _BLK, D), lambda i: (0, i, 0)),
                  pl.BlockSpec((1, _BLK, D), lambda i: (1, i, 0)),
                  pl.BlockSpec((_BLK, D), lambda i: (i, 0)),
                  pl.BlockSpec((_BLK, D), lambda i, k=kcol: (i, k)),
                  pl.BlockSpec((_BLK, 1), lambda i: (i, 0))],
        out_specs=pl.BlockSpec((_BLK, D), lambda i: (i, 0)),
        out_shape=jax.ShapeDtypeStruct((N, D), _f32),
    )(partial, partial, s, Z, selfw2d)


def _comb3_body(p0_ref, p1_ref, s_ref, z_ref, sw_ref, bias_ref, b3_ref,
                o_ref, ssum_ref, ssq_ref):
    o = (p0_ref[0] + p1_ref[0] + sw_ref[...] * s_ref[...] + z_ref[...]
         + bias_ref[0:1, :])
    o_ref[...] = o
    bvec = b3_ref[0]
    gidx = lax.broadcasted_iota(jnp.int32, (G, _BLK), 0)
    onehot = (bvec == gidx).astype(_f32)
    ssum_ref[0] = jnp.dot(onehot, o, preferred_element_type=_f32)
    ssq_ref[0] = jnp.dot(onehot, o * o, preferred_element_type=_f32)


def _combine3_tc(partial, s, Z, selfw2d, bias8, batch3d):
    return pl.pallas_call(
        _comb3_body,
        grid=(_NB,),
        in_specs=[pl.BlockSpec((1, _BLK, D), lambda i: (0, i, 0)),
                  pl.BlockSpec((1, _BLK, D), lambda i: (1, i, 0)),
                  pl.BlockSpec((_BLK, D), lambda i: (i, 0)),
                  pl.BlockSpec((_BLK, D), lambda i: (i, 0)),
                  pl.BlockSpec((_BLK, 1), lambda i: (i, 0)),
                  pl.BlockSpec((8, D), lambda i: (0, 0)),
                  pl.BlockSpec((1, 1, _BLK), lambda i: (i, 0, 0))],
        out_specs=[pl.BlockSpec((_BLK, D), lambda i: (i, 0)),
                   pl.BlockSpec((1, G, D), lambda i: (i, 0, 0)),
                   pl.BlockSpec((1, G, D), lambda i: (i, 0, 0))],
        out_shape=[jax.ShapeDtypeStruct((N, D), _f32),
                   jax.ShapeDtypeStruct((_NB, G, D), _f32),
                   jax.ShapeDtypeStruct((_NB, G, D), _f32)],
    )(partial, partial, s, Z, selfw2d, bias8, batch3d)


def _final_body(hsp_ref, gmp_ref, gm_ref, gx_ref):
    hs = hsp_ref[...]
    gm_ref[...] = hs[0, :G, :] + hs[1, :G, :]
    gx_ref[...] = jnp.max(gmp_ref[...][:, :G, :], axis=0)


def _final_tc(hsp, gmp):
    return pl.pallas_call(
        _final_body,
        out_shape=[jax.ShapeDtypeStruct((G, H), _f32),
                   jax.ShapeDtypeStruct((G, H), _f32)],
    )(hsp, gmp)


@jax.jit
def _impl(inputs, edge_index, batch, edge_weight, W0, W1, W2, W3, conv_bias,
          gn_weight, gn_bias, gn_mean_scale):
    rowi2 = edge_index[0].reshape(NW, EPW)
    coli2 = edge_index[1].reshape(NW, EPW)
    coli3 = edge_index[1].reshape(NW, ECH, EC)
    ew2 = edge_weight.reshape(NW, EPW)
    batch_pad = jnp.pad(batch, (0, NP - N), constant_values=G)
    batch3d = batch.reshape(_NB, 1, _BLK)
    batch2 = batch_pad.reshape(NW, RCH, EC)
    zrows = jnp.zeros((640, D), _f32)

    wcat = jnp.concatenate([W0, W1, W2, W3], axis=1)
    Z = _matmul_z(inputs, wcat)

    degp, cntp = _deg_sc(coli3, ew2, batch2)
    deg = degp[0] + degp[1] + 1.0
    dinv = lax.rsqrt(deg)
    selfw2d = (dinv[:N] * dinv[:N]).reshape(N, 1)
    cnt = jnp.maximum(cntp[0, :G] + cntp[1, :G], 1.0)

    norm2 = _norm_sc(rowi2, coli2, ew2, dinv)

    bias8 = jnp.broadcast_to(conv_bias.reshape(1, H), (8, H))

    s = Z[:, 3 * H:]
    p = _hop_sc(s, rowi2, coli2, norm2, zrows)
    s = _combine_tc(p, s, Z, 2, selfw2d)
    p = _hop_sc(s, rowi2, coli2, norm2, zrows)
    s = _combine_tc(p, s, Z, 1, selfw2d)
    p = _hop_sc(s, rowi2, coli2, norm2, zrows)
    out, ssum_b, ssq_b = _combine3_tc(p, s, Z[:, :H], selfw2d, bias8,
                                      batch3d)

    cnt_c = cnt[:, None]
    mean = jnp.sum(ssum_b, axis=0) / cnt_c
    msc = mean * gn_mean_scale
    var = jnp.sum(ssq_b, axis=0) / cnt_c - 2.0 * msc * mean + msc * msc
    rstd = lax.rsqrt(var + 1e-5)
    Atab = gn_weight * rstd
    Btab = gn_bias - msc * Atab
    Atab = jnp.pad(Atab, ((0, GP - G), (0, 0)))
    Btab = jnp.pad(Btab, ((0, GP - G), (0, 0)))

    out_pad = jnp.pad(out, ((0, NP - N), (0, 0)))
    h_pad, hsp, gmp = _pool_sc(out_pad, batch2, Atab, Btab)
    h = h_pad[:N]

    gmean_sum, gmax_red = _final_tc(hsp, gmp)
    gmean = gmean_sum / cnt_c
    gmax = jnp.where(jnp.isfinite(gmax_red), gmax_red, 0.0)
    flat = jnp.concatenate([gmean, gmax], axis=-1)
    return (h, flat, edge_index, edge_weight, batch)


def kernel(inputs, edge_index, batch, edge_weight, W0, W1, W2, W3, conv_bias,
           gn_weight, gn_bias, gn_mean_scale):
    return _impl(inputs, edge_index, batch, edge_weight, W0, W1, W2, W3,
                 conv_bias, gn_weight, gn_bias, gn_mean_scale)


# speedup vs baseline: 16.4294x; 1.1521x over previous
import jax
import jax.numpy as jnp
from jax import lax
from jax.experimental import pallas as pl
from jax.experimental.pallas import tpu as pltpu
from jax.experimental.pallas import tpu_sc as plsc

N = 10000
NP = 10240
E = 320000
D = 128
H = 128
G = 64
GP = 72
NC = 2
NS = 16
NW = NC * NS
EC = 80
EPW = E // NW
ECH = EPW // EC
RPW = NP // NW
RCH = RPW // EC
RPT = NP // NS
APT = N // NS

_mesh = plsc.VectorSubcoreMesh(core_axis_name="c", subcore_axis_name="s")
_f32 = jnp.float32
_sc_params = pltpu.CompilerParams(needs_layout_passes=False)


def _wid():
    return lax.axis_index("c") * NS + lax.axis_index("s")


def _zero_vec():
    return jnp.zeros((16,), _f32)


def _deg_body(coli3, ew2, batch2, degp, cntp, dacc, cacc, colbuf2, ewbuf,
              onesbuf, bibuf2, zbuf):
    cid = lax.axis_index("c")
    sid = lax.axis_index("s")
    wid = _wid()

    def _z(i, _):
        zbuf[pl.ds(i * 16, 16)] = _zero_vec()
        return 0
    lax.fori_loop(0, RPT // 16, _z, 0)

    def _o(i, _):
        onesbuf[pl.ds(i * 16, 16)] = jnp.ones((16,), _f32)
        return 0
    lax.fori_loop(0, EC // 16, _o, 0)

    pltpu.sync_copy(zbuf, dacc.at[pl.ds(sid * RPT, RPT)])

    @pl.when(sid == 0)
    def _():
        pltpu.sync_copy(zbuf.at[pl.ds(0, GP)], cacc)

    pltpu.sync_copy(coli3.at[wid], colbuf2)
    pltpu.sync_copy(ew2.at[wid], ewbuf)
    pltpu.sync_copy(batch2.at[wid], bibuf2)

    plsc.subcore_barrier()

    def _edges(c, _):
        pltpu.sync_copy(ewbuf.at[pl.ds(c * EC, EC)], dacc.at[colbuf2.at[c]],
                        add=True)
        return 0
    lax.fori_loop(0, ECH, _edges, 0)

    def _nodes(c, _):
        pltpu.sync_copy(onesbuf, cacc.at[bibuf2.at[c]], add=True)
        return 0
    lax.fori_loop(0, RCH, _nodes, 0)

    plsc.subcore_barrier()

    pltpu.sync_copy(dacc.at[pl.ds(sid * RPT, RPT)],
                    degp.at[cid, pl.ds(sid * RPT, RPT)])

    @pl.when(sid == 0)
    def _():
        pltpu.sync_copy(cacc, cntp.at[cid])


_deg_sc = pl.kernel(
    _deg_body,
    out_type=(jax.ShapeDtypeStruct((NC, NP), _f32),
              jax.ShapeDtypeStruct((NC, GP), _f32)),
    mesh=_mesh,
    compiler_params=_sc_params,
    scratch_types=[
        pltpu.VMEM_SHARED((NP,), _f32),
        pltpu.VMEM_SHARED((GP,), _f32),
        pltpu.VMEM((ECH, EC), jnp.int32),
        pltpu.VMEM((EPW,), _f32),
        pltpu.VMEM((EC,), _f32),
        pltpu.VMEM((RCH, EC), jnp.int32),
        pltpu.VMEM((RPT,), _f32),
    ],
)


def _norm_body(rowi2, coli2, ew2, dinv, normo, dvb, rib, cib, nbuf):
    wid = _wid()
    pltpu.sync_copy(dinv, dvb)
    pltpu.sync_copy(rowi2.at[wid], rib)
    pltpu.sync_copy(coli2.at[wid], cib)
    pltpu.sync_copy(ew2.at[wid], nbuf)

    def _chunk(t, _):
        sl = pl.ds(t * 16, 16)
        dr = plsc.load_gather(dvb, [rib[sl]])
        dc = plsc.load_gather(dvb, [cib[sl]])
        nbuf[sl] = dr * nbuf[sl] * dc
        return 0
    lax.fori_loop(0, EPW // 16, _chunk, 0)
    pltpu.sync_copy(nbuf, normo.at[wid])


_norm_sc = pl.kernel(
    _norm_body,
    out_type=jax.ShapeDtypeStruct((NW, EPW), _f32),
    mesh=_mesh,
    compiler_params=_sc_params,
    scratch_types=[
        pltpu.VMEM((NP,), _f32),
        pltpu.VMEM((EPW,), jnp.int32),
        pltpu.VMEM((EPW,), jnp.int32),
        pltpu.VMEM((EPW,), _f32),
    ],
)


HC = 40
HCH = EPW // HC


def _scale_chunk(rows_b, nbuf, c):
    base = c * HC
    nvA = nbuf[pl.ds(base, 16)]
    nvB = nbuf[pl.ds(base + 16, 16)]
    nvC = nbuf[pl.ds(base + 24, 16)]
    for e in range(HC):
        if e < 16:
            nv = nvA[e]
        elif e < 32:
            nv = nvB[e - 16]
        else:
            nv = nvC[e - 24]
        nvec = jnp.full((16,), nv, _f32)
        for j in range(8):
            sl = pl.ds(j * 16, 16)
            rows_b[e, sl] = rows_b[e, sl] * nvec


def _stage_cols(csm, cib, c):
    base = c * HC
    csm[pl.ds(0, 16)] = cib[pl.ds(base, 16)]
    csm[pl.ds(16, 16)] = cib[pl.ds(base + 16, 16)]
    csm[pl.ds(24, 16)] = cib[pl.ds(base + 24, 16)]


def _hop_body(s, rowi2, coli2, norm2, zrows, outp, acc, rib, cib, nbuf,
              csm0, csm1, rows0, rows1, gsem0, gsem1, ssem0, ssem1):
    cid = lax.axis_index("c")
    sid = lax.axis_index("s")
    wid = _wid()

    pltpu.sync_copy(zrows.at[pl.ds(0, 624)], acc.at[pl.ds(sid * 624, 624)])

    @pl.when(sid == NS - 1)
    def _():
        pltpu.sync_copy(zrows.at[pl.ds(624, 16)], acc.at[pl.ds(9984, 16)])

    pltpu.sync_copy(rowi2.at[wid], rib)
    pltpu.sync_copy(coli2.at[wid], cib)
    pltpu.sync_copy(norm2.at[wid], nbuf)

    plsc.subcore_barrier()

    pltpu.async_copy(s.at[rib.at[pl.ds(0, HC)]], rows0, gsem0)
    pltpu.async_copy(s.at[rib.at[pl.ds(HC, HC)]], rows1, gsem1)

    def _pair(cc, _):
        c0 = cc * 2
        c1 = c0 + 1
        pltpu.make_async_copy(s.at[rib.at[pl.ds(c0 * HC, HC)]], rows0,
                              gsem0).wait()
        _scale_chunk(rows0, nbuf, c0)
        _stage_cols(csm0, cib, c0)
        sd0 = pltpu.async_copy(rows0, acc.at[csm0], ssem0, add=True)
        pltpu.make_async_copy(s.at[rib.at[pl.ds(c1 * HC, HC)]], rows1,
                              gsem1).wait()
        _scale_chunk(rows1, nbuf, c1)
        _stage_cols(csm1, cib, c1)
        sd1 = pltpu.async_copy(rows1, acc.at[csm1], ssem1, add=True)
        sd0.wait()

        @pl.when(cc < HCH // 2 - 1)
        def _():
            pltpu.async_copy(s.at[rib.at[pl.ds((c0 + 2) * HC, HC)]], rows0,
                             gsem0)

        sd1.wait()

        @pl.when(cc < HCH // 2 - 1)
        def _():
            pltpu.async_copy(s.at[rib.at[pl.ds((c1 + 2) * HC, HC)]], rows1,
                             gsem1)

        return 0
    lax.fori_loop(0, HCH // 2, _pair, 0)

    plsc.subcore_barrier()
    pltpu.sync_copy(acc.at[pl.ds(sid * 624, 624)],
                    outp.at[cid, pl.ds(sid * 624, 624)])

    @pl.when(sid == NS - 1)
    def _():
        pltpu.sync_copy(acc.at[pl.ds(9984, 16)],
                        outp.at[cid, pl.ds(9984, 16)])


_hop_sc = pl.kernel(
    _hop_body,
    out_type=jax.ShapeDtypeStruct((NC, N, D), _f32),
    mesh=_mesh,
    compiler_params=_sc_params,
    scratch_types=[
        pltpu.VMEM_SHARED((N, D), _f32),
        pltpu.VMEM((EPW,), jnp.int32),
        pltpu.VMEM((EPW,), jnp.int32),
        pltpu.VMEM((EPW,), _f32),
        pltpu.VMEM((HC,), jnp.int32),
        pltpu.VMEM((HC,), jnp.int32),
        pltpu.VMEM((HC, D), _f32),
        pltpu.VMEM((HC, D), _f32),
        pltpu.SemaphoreType.DMA,
        pltpu.SemaphoreType.DMA,
        pltpu.SemaphoreType.DMA,
        pltpu.SemaphoreType.DMA,
    ],
)


def _pool_body(outn, batch2, Atab, Btab, ho, hsp, gmp, hsacc, At, Bt, gmax_t,
               rbuf, hbuf, bibuf2):
    cid = lax.axis_index("c")
    sid = lax.axis_index("s")
    wid = _wid()

    pltpu.sync_copy(Atab, At)
    pltpu.sync_copy(Btab, Bt)
    pltpu.sync_copy(batch2.at[wid], bibuf2)

    def _im(i, _):
        for j in range(8):
            gmax_t[i, pl.ds(j * 16, 16)] = jnp.full((16,), -jnp.inf, _f32)
        return 0
    lax.fori_loop(0, GP, _im, 0)

    def _zh(i, _):
        for j in range(8):
            hbuf[i, pl.ds(j * 16, 16)] = _zero_vec()
        return 0
    lax.fori_loop(0, GP, _zh, 0)

    @pl.when(sid == 0)
    def _():
        pltpu.sync_copy(hbuf.at[pl.ds(0, GP)], hsacc)

    plsc.subcore_barrier()

    def _chunk(c, _):
        base = wid * RPW + c * EC
        pltpu.sync_copy(outn.at[pl.ds(base, EC)], rbuf)

        def _node(t, _):
            b16 = bibuf2[c, pl.ds(t * 16, 16)]
            for e in range(16):
                b = b16[e]
                i = t * 16 + e
                for j in range(8):
                    sl = pl.ds(j * 16, 16)
                    hv = jnp.maximum(rbuf[i, sl] * At[b, sl] + Bt[b, sl],
                                     0.0)
                    hbuf[i, sl] = hv
                    gmax_t[b, sl] = jnp.maximum(gmax_t[b, sl], hv)
            return 0
        lax.fori_loop(0, EC // 16, _node, 0)

        pltpu.sync_copy(hbuf, ho.at[pl.ds(base, EC)])
        pltpu.sync_copy(hbuf, hsacc.at[bibuf2.at[c]], add=True)
        return 0
    lax.fori_loop(0, RCH, _chunk, 0)

    plsc.subcore_barrier()

    @pl.when(sid == 0)
    def _():
        pltpu.sync_copy(hsacc, hsp.at[cid])
    pltpu.sync_copy(gmax_t, gmp.at[wid])


_pool_sc = pl.kernel(
    _pool_body,
    out_type=(jax.ShapeDtypeStruct((NP, H), _f32),
              jax.ShapeDtypeStruct((NC, GP, H), _f32),
              jax.ShapeDtypeStruct((NW, GP, H), _f32)),
    mesh=_mesh,
    compiler_params=_sc_params,
    scratch_types=[
        pltpu.VMEM_SHARED((GP, H), _f32),
        pltpu.VMEM((GP, H), _f32),
        pltpu.VMEM((GP, H), _f32),
        pltpu.VMEM((GP, H), _f32),
        pltpu.VMEM((EC, H), _f32),
        pltpu.VMEM((EC, H), _f32),
        pltpu.VMEM((RCH, EC), jnp.int32),
    ],
)


_BLK = 2000
_NB = N // _BLK


def _z_body(x_ref, w_ref, o_ref):
    o_ref[...] = jnp.dot(x_ref[...], w_ref[...],
                         preferred_element_type=_f32)


def _matmul_z(x, wcat):
    return pl.pallas_call(
        _z_body,
        grid=(_NB,),
        in_specs=[pl.BlockSpec((_BLK, D), lambda i: (i, 0)),
                  pl.BlockSpec((D, 4 * H), lambda i: (0, 0))],
        out_specs=pl.BlockSpec((_BLK, 4 * H), lambda i: (i, 0)),
        out_shape=jax.ShapeDtypeStruct((N, 4 * H), _f32),
    )(x, wcat)


def _comb_body(p0_ref, p1_ref, s_ref, z_ref, sw_ref, o_ref):
    o_ref[...] = (p0_ref[0] + p1_ref[0] + sw_ref[...] * s_ref[...]
                  + z_ref[...])


def _combine_tc(partial, s, Z, kcol, selfw2d):
    return pl.pallas_call(
        _comb_body,
        grid=(_NB,),
        in_specs=[pl.BlockSpec((1, _BLK, D), lambda i: (0, i, 0)),
                  pl.BlockSpec((1, _BLK, D), lambda i: (1, i, 0)),
                  pl.BlockSpec((_BLK, D), lambda i: (i, 0)),
                  pl.BlockSpec((_BLK, D), lambda i, k=kcol: (i, k)),
                  pl.BlockSpec((_BLK, 1), lambda i: (i, 0))],
        out_specs=pl.BlockSpec((_BLK, D), lambda i: (i, 0)),
        out_shape=jax.ShapeDtypeStruct((N, D), _f32),
    )(partial, partial, s, Z, selfw2d)


def _comb3_body(p0_ref, p1_ref, s_ref, z_ref, sw_ref, bias_ref, b3_ref,
                o_ref, ssum_ref, ssq_ref):
    o = (p0_ref[0] + p1_ref[0] + sw_ref[...] * s_ref[...] + z_ref[...]
         + bias_ref[0:1, :])
    o_ref[...] = o
    bvec = b3_ref[0]
    gidx = lax.broadcasted_iota(jnp.int32, (G, _BLK), 0)
    onehot = (bvec == gidx).astype(_f32)
    ssum_ref[0] = jnp.dot(onehot, o, preferred_element_type=_f32)
    ssq_ref[0] = jnp.dot(onehot, o * o, preferred_element_type=_f32)


def _combine3_tc(partial, s, Z, selfw2d, bias8, batch3d):
    return pl.pallas_call(
        _comb3_body,
        grid=(_NB,),
        in_specs=[pl.BlockSpec((1, _BLK, D), lambda i: (0, i, 0)),
                  pl.BlockSpec((1, _BLK, D), lambda i: (1, i, 0)),
                  pl.BlockSpec((_BLK, D), lambda i: (i, 0)),
                  pl.BlockSpec((_BLK, D), lambda i: (i, 0)),
                  pl.BlockSpec((_BLK, 1), lambda i: (i, 0)),
                  pl.BlockSpec((8, D), lambda i: (0, 0)),
                  pl.BlockSpec((1, 1, _BLK), lambda i: (i, 0, 0))],
        out_specs=[pl.BlockSpec((_BLK, D), lambda i: (i, 0)),
                   pl.BlockSpec((1, G, D), lambda i: (i, 0, 0)),
                   pl.BlockSpec((1, G, D), lambda i: (i, 0, 0))],
        out_shape=[jax.ShapeDtypeStruct((N, D), _f32),
                   jax.ShapeDtypeStruct((_NB, G, D), _f32),
                   jax.ShapeDtypeStruct((_NB, G, D), _f32)],
    )(partial, partial, s, Z, selfw2d, bias8, batch3d)


def _final_body(hsp_ref, gmp_ref, gm_ref, gx_ref):
    hs = hsp_ref[...]
    gm_ref[...] = hs[0, :G, :] + hs[1, :G, :]
    gx_ref[...] = jnp.max(gmp_ref[...][:, :G, :], axis=0)


def _final_tc(hsp, gmp):
    return pl.pallas_call(
        _final_body,
        out_shape=[jax.ShapeDtypeStruct((G, H), _f32),
                   jax.ShapeDtypeStruct((G, H), _f32)],
    )(hsp, gmp)


@jax.jit
def _impl(inputs, edge_index, batch, edge_weight, W0, W1, W2, W3, conv_bias,
          gn_weight, gn_bias, gn_mean_scale):
    rowi2 = edge_index[0].reshape(NW, EPW)
    coli2 = edge_index[1].reshape(NW, EPW)
    coli3 = edge_index[1].reshape(NW, ECH, EC)
    ew2 = edge_weight.reshape(NW, EPW)
    batch_pad = jnp.pad(batch, (0, NP - N), constant_values=G)
    batch3d = batch.reshape(_NB, 1, _BLK)
    batch2 = batch_pad.reshape(NW, RCH, EC)
    zrows = jnp.zeros((640, D), _f32)

    wcat = jnp.concatenate([W0, W1, W2, W3], axis=1)
    Z = _matmul_z(inputs, wcat)

    degp, cntp = _deg_sc(coli3, ew2, batch2)
    deg = degp[0] + degp[1] + 1.0
    dinv = lax.rsqrt(deg)
    selfw2d = (dinv[:N] * dinv[:N]).reshape(N, 1)
    cnt = jnp.maximum(cntp[0, :G] + cntp[1, :G], 1.0)

    norm2 = _norm_sc(rowi2, coli2, ew2, dinv)

    bias8 = jnp.broadcast_to(conv_bias.reshape(1, H), (8, H))

    s = Z[:, 3 * H:]
    p = _hop_sc(s, rowi2, coli2, norm2, zrows)
    s = _combine_tc(p, s, Z, 2, selfw2d)
    p = _hop_sc(s, rowi2, coli2, norm2, zrows)
    s = _combine_tc(p, s, Z, 1, selfw2d)
    p = _hop_sc(s, rowi2, coli2, norm2, zrows)
    out, ssum_b, ssq_b = _combine3_tc(p, s, Z[:, :H], selfw2d, bias8,
                                      batch3d)

    cnt_c = cnt[:, None]
    mean = jnp.sum(ssum_b, axis=0) / cnt_c
    msc = mean * gn_mean_scale
    var = jnp.sum(ssq_b, axis=0) / cnt_c - 2.0 * msc * mean + msc * msc
    rstd = lax.rsqrt(var + 1e-5)
    Atab = gn_weight * rstd
    Btab = gn_bias - msc * Atab
    Atab = jnp.pad(Atab, ((0, GP - G), (0, 0)))
    Btab = jnp.pad(Btab, ((0, GP - G), (0, 0)))

    out_pad = jnp.pad(out, ((0, NP - N), (0, 0)))
    h_pad, hsp, gmp = _pool_sc(out_pad, batch2, Atab, Btab)
    h = h_pad[:N]

    gmean_sum, gmax_red = _final_tc(hsp, gmp)
    gmean = gmean_sum / cnt_c
    gmax = jnp.where(jnp.isfinite(gmax_red), gmax_red, 0.0)
    flat = jnp.concatenate([gmean, gmax], axis=-1)
    return (h, flat, edge_index, edge_weight, batch)


def kernel(inputs, edge_index, batch, edge_weight, W0, W1, W2, W3, conv_bias,
           gn_weight, gn_bias, gn_mean_scale):
    return _impl(inputs, edge_index, batch, edge_weight, W0, W1, W2, W3,
                 conv_bias, gn_weight, gn_bias, gn_mean_scale)


# speedup vs baseline: 16.4458x; 1.0010x over previous
import jax
import jax.numpy as jnp
from jax import lax
from jax.experimental import pallas as pl
from jax.experimental.pallas import tpu as pltpu
from jax.experimental.pallas import tpu_sc as plsc

N = 10000
NP = 10240
E = 320000
D = 128
H = 128
G = 64
GP = 72
NC = 2
NS = 16
NW = NC * NS
EC = 80
EPW = E // NW
ECH = EPW // EC
RPW = NP // NW
RCH = RPW // EC
RPT = NP // NS

_mesh = plsc.VectorSubcoreMesh(core_axis_name="c", subcore_axis_name="s")
_f32 = jnp.float32
_sc_params = pltpu.CompilerParams(needs_layout_passes=False)


def _wid():
    return lax.axis_index("c") * NS + lax.axis_index("s")


def _zero_vec():
    return jnp.zeros((16,), _f32)


def _deg_body(coli3, ew2, batch2, degp, cntp, dacc, cacc, colbuf2, ewbuf,
              onesbuf, bibuf2, zbuf):
    cid = lax.axis_index("c")
    sid = lax.axis_index("s")
    wid = _wid()

    def _z(i, _):
        zbuf[pl.ds(i * 16, 16)] = _zero_vec()
        return 0
    lax.fori_loop(0, RPT // 16, _z, 0)

    def _o(i, _):
        onesbuf[pl.ds(i * 16, 16)] = jnp.ones((16,), _f32)
        return 0
    lax.fori_loop(0, EC // 16, _o, 0)

    pltpu.sync_copy(zbuf, dacc.at[pl.ds(sid * RPT, RPT)])

    @pl.when(sid == 0)
    def _():
        pltpu.sync_copy(zbuf.at[pl.ds(0, GP)], cacc)

    pltpu.sync_copy(coli3.at[wid], colbuf2)
    pltpu.sync_copy(ew2.at[wid], ewbuf)
    pltpu.sync_copy(batch2.at[wid], bibuf2)

    plsc.subcore_barrier()

    def _edges(c, _):
        pltpu.sync_copy(ewbuf.at[pl.ds(c * EC, EC)], dacc.at[colbuf2.at[c]],
                        add=True)
        return 0
    lax.fori_loop(0, ECH, _edges, 0)

    def _nodes(c, _):
        pltpu.sync_copy(onesbuf, cacc.at[bibuf2.at[c]], add=True)
        return 0
    lax.fori_loop(0, RCH, _nodes, 0)

    plsc.subcore_barrier()

    pltpu.sync_copy(dacc.at[pl.ds(sid * RPT, RPT)],
                    degp.at[cid, pl.ds(sid * RPT, RPT)])

    @pl.when(sid == 0)
    def _():
        pltpu.sync_copy(cacc, cntp.at[cid])


_deg_sc = pl.kernel(
    _deg_body,
    out_type=(jax.ShapeDtypeStruct((NC, NP), _f32),
              jax.ShapeDtypeStruct((NC, GP), _f32)),
    mesh=_mesh,
    compiler_params=_sc_params,
    scratch_types=[
        pltpu.VMEM_SHARED((NP,), _f32),
        pltpu.VMEM_SHARED((GP,), _f32),
        pltpu.VMEM((ECH, EC), jnp.int32),
        pltpu.VMEM((EPW,), _f32),
        pltpu.VMEM((EC,), _f32),
        pltpu.VMEM((RCH, EC), jnp.int32),
        pltpu.VMEM((RPT,), _f32),
    ],
)


def _norm_body(rowi2, coli2, ew2, dinv, normo, dvb, rib, cib, nbuf):
    wid = _wid()
    pltpu.sync_copy(dinv, dvb)
    pltpu.sync_copy(rowi2.at[wid], rib)
    pltpu.sync_copy(coli2.at[wid], cib)
    pltpu.sync_copy(ew2.at[wid], nbuf)

    def _chunk(t, _):
        sl = pl.ds(t * 16, 16)
        dr = plsc.load_gather(dvb, [rib[sl]])
        dc = plsc.load_gather(dvb, [cib[sl]])
        nbuf[sl] = dr * nbuf[sl] * dc
        return 0
    lax.fori_loop(0, EPW // 16, _chunk, 0)
    pltpu.sync_copy(nbuf, normo.at[wid])


_norm_sc = pl.kernel(
    _norm_body,
    out_type=jax.ShapeDtypeStruct((NW, EPW), _f32),
    mesh=_mesh,
    compiler_params=_sc_params,
    scratch_types=[
        pltpu.VMEM((NP,), _f32),
        pltpu.VMEM((EPW,), jnp.int32),
        pltpu.VMEM((EPW,), jnp.int32),
        pltpu.VMEM((EPW,), _f32),
    ],
)


HC = 40
HCH = EPW // HC


def _scale_chunk(rows_b, nbuf, c):
    base = c * HC
    nvA = nbuf[pl.ds(base, 16)]
    nvB = nbuf[pl.ds(base + 16, 16)]
    nvC = nbuf[pl.ds(base + 24, 16)]
    for e in range(HC):
        if e < 16:
            nv = nvA[e]
        elif e < 32:
            nv = nvB[e - 16]
        else:
            nv = nvC[e - 24]
        nvec = jnp.full((16,), nv, _f32)
        for j in range(8):
            sl = pl.ds(j * 16, 16)
            rows_b[e, sl] = rows_b[e, sl] * nvec


def _stage_cols(csm, cib, c):
    base = c * HC
    csm[pl.ds(0, 16)] = cib[pl.ds(base, 16)]
    csm[pl.ds(16, 16)] = cib[pl.ds(base + 16, 16)]
    csm[pl.ds(24, 16)] = cib[pl.ds(base + 24, 16)]


def _hop_body(s, rowi2, coli2, norm2, zrows, outp, acc, rib, cib, nbuf,
              csm0, csm1, rows0, rows1, gsem0, gsem1, ssem0, ssem1):
    cid = lax.axis_index("c")
    sid = lax.axis_index("s")
    wid = _wid()

    pltpu.sync_copy(zrows.at[pl.ds(0, 624)], acc.at[pl.ds(sid * 624, 624)])

    @pl.when(sid == NS - 1)
    def _():
        pltpu.sync_copy(zrows.at[pl.ds(624, 16)], acc.at[pl.ds(9984, 16)])

    pltpu.sync_copy(rowi2.at[wid], rib)
    pltpu.sync_copy(coli2.at[wid], cib)
    pltpu.sync_copy(norm2.at[wid], nbuf)

    plsc.subcore_barrier()

    pltpu.async_copy(s.at[rib.at[pl.ds(0, HC)]], rows0, gsem0)
    pltpu.async_copy(s.at[rib.at[pl.ds(HC, HC)]], rows1, gsem1)

    def _pair(cc, _):
        c0 = cc * 2
        c1 = c0 + 1
        pltpu.make_async_copy(s.at[rib.at[pl.ds(c0 * HC, HC)]], rows0,
                              gsem0).wait()
        _scale_chunk(rows0, nbuf, c0)
        _stage_cols(csm0, cib, c0)
        sd0 = pltpu.async_copy(rows0, acc.at[csm0], ssem0, add=True)
        pltpu.make_async_copy(s.at[rib.at[pl.ds(c1 * HC, HC)]], rows1,
                              gsem1).wait()
        _scale_chunk(rows1, nbuf, c1)
        _stage_cols(csm1, cib, c1)
        sd1 = pltpu.async_copy(rows1, acc.at[csm1], ssem1, add=True)
        sd0.wait()

        @pl.when(cc < HCH // 2 - 1)
        def _():
            pltpu.async_copy(s.at[rib.at[pl.ds((c0 + 2) * HC, HC)]], rows0,
                             gsem0)

        sd1.wait()

        @pl.when(cc < HCH // 2 - 1)
        def _():
            pltpu.async_copy(s.at[rib.at[pl.ds((c1 + 2) * HC, HC)]], rows1,
                             gsem1)

        return 0
    lax.fori_loop(0, HCH // 2, _pair, 0)

    plsc.subcore_barrier()
    pltpu.sync_copy(acc.at[pl.ds(sid * 624, 624)],
                    outp.at[cid, pl.ds(sid * 624, 624)])

    @pl.when(sid == NS - 1)
    def _():
        pltpu.sync_copy(acc.at[pl.ds(9984, 16)],
                        outp.at[cid, pl.ds(9984, 16)])


_hop_sc = pl.kernel(
    _hop_body,
    out_type=jax.ShapeDtypeStruct((NC, N, D), _f32),
    mesh=_mesh,
    compiler_params=_sc_params,
    scratch_types=[
        pltpu.VMEM_SHARED((N, D), _f32),
        pltpu.VMEM((EPW,), jnp.int32),
        pltpu.VMEM((EPW,), jnp.int32),
        pltpu.VMEM((EPW,), _f32),
        pltpu.VMEM((HC,), jnp.int32),
        pltpu.VMEM((HC,), jnp.int32),
        pltpu.VMEM((HC, D), _f32),
        pltpu.VMEM((HC, D), _f32),
        pltpu.SemaphoreType.DMA,
        pltpu.SemaphoreType.DMA,
        pltpu.SemaphoreType.DMA,
        pltpu.SemaphoreType.DMA,
    ],
)


def _pool_body(outn, batch2, Atab, Btab, ho, hsp, gmp, hsacc, At, Bt, gmax_t,
               rbuf, hbuf, bibuf2):
    cid = lax.axis_index("c")
    sid = lax.axis_index("s")
    wid = _wid()

    pltpu.sync_copy(Atab, At)
    pltpu.sync_copy(Btab, Bt)
    pltpu.sync_copy(batch2.at[wid], bibuf2)

    def _im(i, _):
        for j in range(8):
            gmax_t[i, pl.ds(j * 16, 16)] = jnp.full((16,), -jnp.inf, _f32)
        return 0
    lax.fori_loop(0, GP, _im, 0)

    def _zh(i, _):
        for j in range(8):
            hbuf[i, pl.ds(j * 16, 16)] = _zero_vec()
        return 0
    lax.fori_loop(0, GP, _zh, 0)

    @pl.when(sid == 0)
    def _():
        pltpu.sync_copy(hbuf.at[pl.ds(0, GP)], hsacc)

    plsc.subcore_barrier()

    def _chunk(c, _):
        base = wid * RPW + c * EC
        pltpu.sync_copy(outn.at[pl.ds(base, EC)], rbuf)

        def _node(t, _):
            b16 = bibuf2[c, pl.ds(t * 16, 16)]
            for e in range(16):
                b = b16[e]
                i = t * 16 + e
                for j in range(8):
                    sl = pl.ds(j * 16, 16)
                    hv = jnp.maximum(rbuf[i, sl] * At[b, sl] + Bt[b, sl],
                                     0.0)
                    hbuf[i, sl] = hv
                    gmax_t[b, sl] = jnp.maximum(gmax_t[b, sl], hv)
            return 0
        lax.fori_loop(0, EC // 16, _node, 0)

        pltpu.sync_copy(hbuf, ho.at[pl.ds(base, EC)])
        pltpu.sync_copy(hbuf, hsacc.at[bibuf2.at[c]], add=True)
        return 0
    lax.fori_loop(0, RCH, _chunk, 0)

    plsc.subcore_barrier()

    @pl.when(sid == 0)
    def _():
        pltpu.sync_copy(hsacc, hsp.at[cid])
    pltpu.sync_copy(gmax_t, gmp.at[wid])


_pool_sc = pl.kernel(
    _pool_body,
    out_type=(jax.ShapeDtypeStruct((NP, H), _f32),
              jax.ShapeDtypeStruct((NC, GP, H), _f32),
              jax.ShapeDtypeStruct((NW, GP, H), _f32)),
    mesh=_mesh,
    compiler_params=_sc_params,
    scratch_types=[
        pltpu.VMEM_SHARED((GP, H), _f32),
        pltpu.VMEM((GP, H), _f32),
        pltpu.VMEM((GP, H), _f32),
        pltpu.VMEM((GP, H), _f32),
        pltpu.VMEM((EC, H), _f32),
        pltpu.VMEM((EC, H), _f32),
        pltpu.VMEM((RCH, EC), jnp.int32),
    ],
)


_BLK = 2000
_NB = N // _BLK


def _z_body(x_ref, w_ref, o_ref):
    o_ref[...] = jnp.dot(x_ref[...], w_ref[...],
                         preferred_element_type=_f32)


def _matmul_z(x, wcat):
    return pl.pallas_call(
        _z_body,
        grid=(_NB,),
        in_specs=[pl.BlockSpec((_BLK, D), lambda i: (i, 0)),
                  pl.BlockSpec((D, 4 * H), lambda i: (0, 0))],
        out_specs=pl.BlockSpec((_BLK, 4 * H), lambda i: (i, 0)),
        out_shape=jax.ShapeDtypeStruct((N, 4 * H), _f32),
    )(x, wcat)


def _comb_body(p0_ref, p1_ref, s_ref, z_ref, sw_ref, o_ref):
    o_ref[...] = (p0_ref[0] + p1_ref[0] + sw_ref[...] * s_ref[...]
                  + z_ref[...])


def _combine_tc(partial, s, Z, kcol, selfw2d):
    return pl.pallas_call(
        _comb_body,
        grid=(_NB,),
        in_specs=[pl.BlockSpec((1, _BLK, D), lambda i: (0, i, 0)),
                  pl.BlockSpec((1, _BLK, D), lambda i: (1, i, 0)),
                  pl.BlockSpec((_BLK, D), lambda i: (i, 0)),
                  pl.BlockSpec((_BLK, D), lambda i, k=kcol: (i, k)),
                  pl.BlockSpec((_BLK, 1), lambda i: (i, 0))],
        out_specs=pl.BlockSpec((_BLK, D), lambda i: (i, 0)),
        out_shape=jax.ShapeDtypeStruct((N, D), _f32),
    )(partial, partial, s, Z, selfw2d)


def _comb3_body(p0_ref, p1_ref, s_ref, z_ref, sw_ref, bias_ref, b3_ref,
                o_ref, ssum_ref, ssq_ref):
    o = (p0_ref[0] + p1_ref[0] + sw_ref[...] * s_ref[...] + z_ref[...]
         + bias_ref[0:1, :])
    o_ref[...] = o
    bvec = b3_ref[0]
    gidx = lax.broadcasted_iota(jnp.int32, (G, _BLK), 0)
    onehot = (bvec == gidx).astype(_f32)
    ssum_ref[0] = jnp.dot(onehot, o, preferred_element_type=_f32)
    ssq_ref[0] = jnp.dot(onehot, o * o, preferred_element_type=_f32)


def _combine3_tc(partial, s, Z, selfw2d, bias8, batch3d):
    return pl.pallas_call(
        _comb3_body,
        grid=(_NB,),
        in_specs=[pl.BlockSpec((1, _BLK, D), lambda i: (0, i, 0)),
                  pl.BlockSpec((1, _BLK, D), lambda i: (1, i, 0)),
                  pl.BlockSpec((_BLK, D), lambda i: (i, 0)),
                  pl.BlockSpec((_BLK, D), lambda i: (i, 0)),
                  pl.BlockSpec((_BLK, 1), lambda i: (i, 0)),
                  pl.BlockSpec((8, D), lambda i: (0, 0)),
                  pl.BlockSpec((1, 1, _BLK), lambda i: (i, 0, 0))],
        out_specs=[pl.BlockSpec((_BLK, D), lambda i: (i, 0)),
                   pl.BlockSpec((1, G, D), lambda i: (i, 0, 0)),
                   pl.BlockSpec((1, G, D), lambda i: (i, 0, 0))],
        out_shape=[jax.ShapeDtypeStruct((N, D), _f32),
                   jax.ShapeDtypeStruct((_NB, G, D), _f32),
                   jax.ShapeDtypeStruct((_NB, G, D), _f32)],
    )(partial, partial, s, Z, selfw2d, bias8, batch3d)


def _final_body(hsp_ref, gmp_ref, gm_ref, gx_ref):
    hs = hsp_ref[...]
    gm_ref[...] = hs[0, :G, :] + hs[1, :G, :]
    gx_ref[...] = jnp.max(gmp_ref[...][:, :G, :], axis=0)


def _final_tc(hsp, gmp):
    return pl.pallas_call(
        _final_body,
        out_shape=[jax.ShapeDtypeStruct((G, H), _f32),
                   jax.ShapeDtypeStruct((G, H), _f32)],
    )(hsp, gmp)


@jax.jit
def _impl(inputs, edge_index, batch, edge_weight, W0, W1, W2, W3, conv_bias,
          gn_weight, gn_bias, gn_mean_scale):
    rowi2 = edge_index[0].reshape(NW, EPW)
    coli2 = edge_index[1].reshape(NW, EPW)
    coli3 = edge_index[1].reshape(NW, ECH, EC)
    ew2 = edge_weight.reshape(NW, EPW)
    batch_pad = jnp.pad(batch, (0, NP - N), constant_values=G)
    batch3d = batch.reshape(_NB, 1, _BLK)
    batch2 = batch_pad.reshape(NW, RCH, EC)
    zrows = jnp.zeros((640, D), _f32)

    wcat = jnp.concatenate([W0, W1, W2, W3], axis=1)
    Z = _matmul_z(inputs, wcat)

    degp, cntp = _deg_sc(coli3, ew2, batch2)
    deg = degp[0] + degp[1] + 1.0
    dinv = lax.rsqrt(deg)
    selfw2d = (dinv[:N] * dinv[:N]).reshape(N, 1)
    cnt = jnp.maximum(cntp[0, :G] + cntp[1, :G], 1.0)

    norm2 = _norm_sc(rowi2, coli2, ew2, dinv)

    bias8 = jnp.broadcast_to(conv_bias.reshape(1, H), (8, H))

    s = Z[:, 3 * H:]
    p = _hop_sc(s, rowi2, coli2, norm2, zrows)
    s = _combine_tc(p, s, Z, 2, selfw2d)
    p = _hop_sc(s, rowi2, coli2, norm2, zrows)
    s = _combine_tc(p, s, Z, 1, selfw2d)
    p = _hop_sc(s, rowi2, coli2, norm2, zrows)
    out, ssum_b, ssq_b = _combine3_tc(p, s, Z[:, :H], selfw2d, bias8,
                                      batch3d)

    cnt_c = cnt[:, None]
    mean = jnp.sum(ssum_b, axis=0) / cnt_c
    msc = mean * gn_mean_scale
    var = jnp.sum(ssq_b, axis=0) / cnt_c - 2.0 * msc * mean + msc * msc
    rstd = lax.rsqrt(var + 1e-5)
    Atab = gn_weight * rstd
    Btab = gn_bias - msc * Atab
    Atab = jnp.pad(Atab, ((0, GP - G), (0, 0)))
    Btab = jnp.pad(Btab, ((0, GP - G), (0, 0)))

    out_pad = jnp.pad(out, ((0, NP - N), (0, 0)))
    h_pad, hsp, gmp = _pool_sc(out_pad, batch2, Atab, Btab)
    h = h_pad[:N]

    gmean_sum, gmax_red = _final_tc(hsp, gmp)
    gmean = gmean_sum / cnt_c
    gmax = jnp.where(jnp.isfinite(gmax_red), gmax_red, 0.0)
    flat = jnp.concatenate([gmean, gmax], axis=-1)
    return (h, flat, edge_index, edge_weight, batch)


def kernel(inputs, edge_index, batch, edge_weight, W0, W1, W2, W3, conv_bias,
           gn_weight, gn_bias, gn_mean_scale):
    return _impl(inputs, edge_index, batch, edge_weight, W0, W1, W2, W3,
                 conv_bias, gn_weight, gn_bias, gn_mean_scale)
